# baseline (speedup 1.0000x reference)
import functools

import jax
import jax.numpy as jnp
from jax import lax
from jax.experimental import pallas as pl
from jax.experimental.pallas import tpu as pltpu
from jax.experimental.pallas import tpu_sc as plsc

B = 64
L = 2048
D = 128

_NC = 2
_NS = 16
_NW = _NC * _NS
_BPW = B // _NW


def _pe_lookup(offsets, pe_flat):
    mesh = plsc.VectorSubcoreMesh(core_axis_name="c", subcore_axis_name="s")

    @functools.partial(
        pl.kernel,
        mesh=mesh,
        out_type=jax.ShapeDtypeStruct((B * L * D,), jnp.float32),
        scratch_types=[
            pltpu.VMEM((B,), jnp.int32),
            pltpu.SemaphoreType.DMA,
        ],
    )
    def k(offs_hbm, pe_hbm, out_hbm, offs_v, sem):
        wid = lax.axis_index("s") * _NC + lax.axis_index("c")
        pltpu.sync_copy(offs_hbm, offs_v)
        for w in range(_NW):

            @pl.when(wid == w)
            def _():
                copies = []
                for j in range(_BPW):
                    b = w * _BPW + j
                    vec = offs_v[pl.ds((b // 16) * 16, 16)]
                    off = vec[b % 16]
                    copies.append(
                        pltpu.async_copy(
                            pe_hbm.at[pl.ds(off * D, L * D)],
                            out_hbm.at[pl.ds(b * (L * D), L * D)],
                            sem,
                        )
                    )
                for cp in copies:
                    cp.wait()

    return k(offsets, pe_flat)


def kernel(x, pe):
    offsets = x[:, 0, 0].astype(jnp.int32)
    flat = _pe_lookup(offsets, pe.reshape(-1))
    return flat.reshape(B, L, D)


# speedup vs baseline: 26.9155x; 26.9155x over previous
import functools

import jax
import jax.numpy as jnp
from jax import lax
from jax.experimental import pallas as pl
from jax.experimental.pallas import tpu as pltpu
from jax.experimental.pallas import tpu_sc as plsc

B = 64
L = 2048
D = 128

_NC = 2
_NS = 16
_NW = _NC * _NS
_BPW = B // _NW
_WORDS = L * D
_CH = 32768
_NCHUNK = _WORDS // _CH


def _copy_batch(pe_hbm, out_hbm, off, b, bufs, sems_in, sems_out):
    src0 = off * D
    dst0 = b * _WORDS

    def in_copy(i, k):
        return pltpu.async_copy(
            pe_hbm.at[pl.ds(src0 + i * _CH, _CH)], bufs[k], sems_in[k]
        )

    def out_copy(i, k):
        return pltpu.async_copy(
            bufs[k], out_hbm.at[pl.ds(dst0 + i * _CH, _CH)], sems_out[k]
        )

    out_pending = [None, None]
    cp_in = in_copy(0, 0)
    for i in range(_NCHUNK):
        k = i % 2
        cp_in.wait()
        if i + 1 < _NCHUNK:
            k2 = (i + 1) % 2
            if out_pending[k2] is not None:
                out_pending[k2].wait()
                out_pending[k2] = None
            cp_in = in_copy(i + 1, k2)
        out_pending[k] = out_copy(i, k)
    for cp in out_pending:
        if cp is not None:
            cp.wait()


def _pe_lookup(offsets, pe_flat):
    mesh = plsc.VectorSubcoreMesh(core_axis_name="c", subcore_axis_name="s")

    @functools.partial(
        pl.kernel,
        mesh=mesh,
        out_type=jax.ShapeDtypeStruct((B * _WORDS,), jnp.float32),
        scratch_types=[
            pltpu.VMEM((B,), jnp.int32),
            pltpu.VMEM((_CH,), jnp.float32),
            pltpu.VMEM((_CH,), jnp.float32),
            pltpu.SemaphoreType.DMA,
            pltpu.SemaphoreType.DMA,
            pltpu.SemaphoreType.DMA,
            pltpu.SemaphoreType.DMA,
        ],
    )
    def k(offs_hbm, pe_hbm, out_hbm, offs_v, buf0, buf1, si0, si1, so0, so1):
        wid = lax.axis_index("s") * _NC + lax.axis_index("c")
        pltpu.sync_copy(offs_hbm, offs_v)
        for w in range(_NW):

            @pl.when(wid == w)
            def _():
                for j in range(_BPW):
                    b = w * _BPW + j
                    vec = offs_v[pl.ds((b // 16) * 16, 16)]
                    off = vec[b % 16]
                    _copy_batch(
                        pe_hbm, out_hbm, off, b,
                        (buf0, buf1), (si0, si1), (so0, so1),
                    )

    return k(offsets, pe_flat)


def kernel(x, pe):
    offsets = x[:, 0, 0].astype(jnp.int32)
    flat = _pe_lookup(offsets, pe.reshape(-1))
    return flat.reshape(B, L, D)
